# baseline (speedup 1.0000x reference)
import functools
import jax
import jax.numpy as jnp
from jax import lax
from jax.experimental import pallas as pl
from jax.experimental.pallas import tpu as pltpu

N = 50000
E = 800000
H = 64
QW = 16
L = 6
ROWS = 400
GRID = N // ROWS

CH = 128
G = 4
EPT = E // 16
NG = EPT // (CH * G)
TREM = EPT - NG * CH * G
TCH = [CH, CH, TREM - 2 * CH]
NP16 = 3128
NPAD = 16 * NP16
BUFCH = [(0, 784), (784, 784), (1568, 784), (2352, 776)]

_HI = lax.Precision.HIGHEST


def _row_spec(f):
    return pl.BlockSpec((ROWS, f), lambda i: (i, 0))


def _full_spec(shape):
    nd = len(shape)
    return pl.BlockSpec(shape, lambda i: (0,) * nd)


def _silu(v):
    return v * jax.nn.sigmoid(v)


def _q_outs():
    return ([_row_spec(QW)] * 8,
            [jax.ShapeDtypeStruct((N, QW), jnp.float32)] * 8)


def _split_proj(ab, out_refs):
    for q in range(8):
        out_refs[q][...] = ab[:, q * QW:(q + 1) * QW]


def _enc_body(x_ref, We1_ref, be1_ref, We2_ref, be2_ref, Wab_ref, b1_ref,
              h_ref, *q_refs):
    h1 = _silu(jnp.dot(x_ref[...], We1_ref[...],
                       preferred_element_type=jnp.float32, precision=_HI) + be1_ref[...])
    h = jnp.dot(h1, We2_ref[...], preferred_element_type=jnp.float32,
                precision=_HI) + be2_ref[...]
    h_ref[...] = h
    ab = jnp.dot(h, Wab_ref[...], preferred_element_type=jnp.float32,
                 precision=_HI) + b1_ref[...]
    _split_proj(ab, q_refs)


def _upd_proj_body(h_ref, S0, S1, S2, S3, deg_ref, W2_ref, b2_ref, Wab_ref,
                   b1_ref, h_ref_o, *q_refs):
    S = jnp.concatenate([S0[...], S1[...], S2[...], S3[...]], axis=-1)
    dcol = deg_ref[:, 0:1]
    h = h_ref[...] + jnp.dot(S, W2_ref[...], preferred_element_type=jnp.float32,
                             precision=_HI) + dcol * b2_ref[...]
    h_ref_o[...] = h
    ab = jnp.dot(h, Wab_ref[...], preferred_element_type=jnp.float32,
                 precision=_HI) + b1_ref[...]
    _split_proj(ab, q_refs)


def _upd_dec_body(h_ref, S0, S1, S2, S3, deg_ref, W2_ref, b2_ref, Wd1_ref,
                  bd1_ref, Wd2_ref, bd2_ref, Wd3_ref, bd3_ref, out_ref):
    S = jnp.concatenate([S0[...], S1[...], S2[...], S3[...]], axis=-1)
    dcol = deg_ref[:, 0:1]
    h = h_ref[...] + jnp.dot(S, W2_ref[...], preferred_element_type=jnp.float32,
                             precision=_HI) + dcol * b2_ref[...]
    o = _silu(jnp.dot(h, Wd1_ref[...], preferred_element_type=jnp.float32,
                      precision=_HI) + bd1_ref[...])
    o = _silu(jnp.dot(o, Wd2_ref[...], preferred_element_type=jnp.float32,
                      precision=_HI) + bd2_ref[...])
    out_ref[...] = jnp.dot(o, Wd3_ref[...], preferred_element_type=jnp.float32,
                           precision=_HI) + bd3_ref[...]


def _enc_call(x, We1, be1, We2, be2, Wab, b1):
    qspecs, qshapes = _q_outs()
    return pl.pallas_call(
        _enc_body,
        grid=(GRID,),
        in_specs=[_row_spec(14), _full_spec((14, H)), _full_spec((1, H)),
                  _full_spec((H, H)), _full_spec((1, H)),
                  _full_spec((H, 2 * H)), _full_spec((1, 2 * H))],
        out_specs=[_row_spec(H)] + qspecs,
        out_shape=[jax.ShapeDtypeStruct((N, H), jnp.float32)] + qshapes,
    )(x, We1, be1, We2, be2, Wab, b1)


def _upd_proj_call(h, S, deg, W2, b2, Wab, b1):
    qspecs, qshapes = _q_outs()
    return pl.pallas_call(
        _upd_proj_body,
        grid=(GRID,),
        in_specs=[_row_spec(H)] + [_row_spec(QW)] * 4 + [_row_spec(16),
                  _full_spec((H, H)), _full_spec((1, H)),
                  _full_spec((H, 2 * H)), _full_spec((1, 2 * H))],
        out_specs=[_row_spec(H)] + qspecs,
        out_shape=[jax.ShapeDtypeStruct((N, H), jnp.float32)] + qshapes,
    )(h, S[0], S[1], S[2], S[3], deg, W2, b2, Wab, b1)


def _upd_dec_call(h, S, deg, W2, b2, Wd1, bd1, Wd2, bd2, Wd3, bd3):
    return pl.pallas_call(
        _upd_dec_body,
        grid=(GRID,),
        in_specs=[_row_spec(H)] + [_row_spec(QW)] * 4 + [_row_spec(16),
                  _full_spec((H, H)), _full_spec((1, H)),
                  _full_spec((H, H)), _full_spec((1, H)),
                  _full_spec((H, H // 2)), _full_spec((1, H // 2)),
                  _full_spec((H // 2, 9)), _full_spec((1, 9))],
        out_specs=[_row_spec(9)],
        out_shape=[jax.ShapeDtypeStruct((N, 9), jnp.float32)],
    )(h, S[0], S[1], S[2], S[3], deg, W2, b2, Wd1, bd1, Wd2, bd2, Wd3, bd3)[0]


BE = 6400


def _prep_body(ei_ref, out_ref):
    out_ref[...] = ei_ref[...]


def _prep_call(edge_index):
    return pl.pallas_call(
        _prep_body,
        grid=(E // BE,),
        in_specs=[pl.BlockSpec((2, BE), lambda i: (0, i))],
        out_specs=[pl.BlockSpec((2, BE), lambda i: (0, i))],
        out_shape=[jax.ShapeDtypeStruct((2, E), jnp.int32)],
    )(edge_index)[0]


def _p_body(ea_ref, w_ref, p_ref):
    p_ref[...] = jnp.dot(ea_ref[...], w_ref[...],
                         preferred_element_type=jnp.float32, precision=_HI)


def _p_call(edge_attr, w):
    return pl.pallas_call(
        _p_body,
        grid=(E // BE,),
        in_specs=[pl.BlockSpec((BE, 3), lambda i: (i, 0)), _full_spec((3, H))],
        out_specs=[pl.BlockSpec((BE, H), lambda i: (i, 0))],
        out_shape=[jax.ShapeDtypeStruct((E, H), jnp.float32)],
    )(edge_attr, w)[0]


_sc_mesh = None


def _mesh():
    global _sc_mesh
    if _sc_mesh is None:
        from jax.experimental.pallas import tpu_sc as plsc
        _sc_mesh = plsc.VectorSubcoreMesh(core_axis_name="c", subcore_axis_name="s")
    return _sc_mesh


def _zero_rows(buf, nrows):
    z = jnp.zeros((16,), jnp.float32)

    def zb(i, _):
        buf[i, pl.ds(0, 16)] = z
        return 0
    lax.fori_loop(0, nrows, zb, 0)


def _edge_phase(sid, Ah, Bh, Ph, qoff, Sout3, ei2,
                didx4, sidx4, didxT, sidxT, Ar4, Br4, Pr4, Mv4, buf,
                Ssh, sld, sga, ssc, drain_src):
    from jax.experimental.pallas import tpu_sc as plsc
    _zero_rows(buf, 784)
    base0 = sid * NP16
    for off, sz in BUFCH:
        pltpu.sync_copy(buf.at[pl.ds(0, sz)], Ssh.at[pl.ds(base0 + off, sz)])
    plsc.subcore_barrier()

    tbase = sid * EPT

    def compute_chunk(Arr, Brr, Prr, Mvr, nch):
        def edge_body(e, _):
            pre = Arr[e, pl.ds(0, 16)] + Brr[e, pl.ds(0, 16)] + Prr[e, pl.ds(0, 16)]
            Mvr[e, pl.ds(0, 16)] = pre / (1.0 + jnp.exp(-pre))
            return 0
        lax.fori_loop(0, nch, edge_body, 0)

    def drain_sc(p):
        pltpu.make_async_copy(drain_src, Mv4.at[p], ssc.at[p]).wait()

    def group_body(g, _):
        gbase = tbase + g * (G * CH)
        lds = []
        for p in range(G):
            eb = gbase + p * CH

            @pl.when(g > 0)
            def _():
                drain_sc(p)
            c0 = pltpu.async_copy(ei2.at[1, pl.ds(eb, CH)], didx4.at[p], sld.at[p])
            c1 = pltpu.async_copy(ei2.at[0, pl.ds(eb, CH)], sidx4.at[p], sld.at[p])
            c2 = pltpu.async_copy(Ph.at[pl.ds(eb, CH), pl.ds(qoff, QW)],
                                  Pr4.at[p], sld.at[p])
            lds.append((c0, c1, c2))
        gas = []
        for p in range(G):
            for c in lds[p]:
                c.wait()
            ga = pltpu.async_copy(Ah.at[didx4.at[p]], Ar4.at[p], sga.at[p])
            gb = pltpu.async_copy(Bh.at[sidx4.at[p]], Br4.at[p], sga.at[p])
            gas.append((ga, gb))
        for p in range(G):
            for c in gas[p]:
                c.wait()
            compute_chunk(Ar4.at[p], Br4.at[p], Pr4.at[p], Mv4.at[p], CH)
            pltpu.async_copy(Mv4.at[p], Ssh.at[didx4.at[p]], ssc.at[p], add=True)
        return 0
    lax.fori_loop(0, NG, group_body, 0)
    for p in range(G):
        drain_sc(p)

    toff = tbase + NG * G * CH
    for i, tch in enumerate(TCH):
        eb = toff + sum(TCH[:i])
        pltpu.sync_copy(ei2.at[1, pl.ds(eb, tch)], didxT.at[pl.ds(0, tch)])
        pltpu.sync_copy(ei2.at[0, pl.ds(eb, tch)], sidxT.at[pl.ds(0, tch)])
        pltpu.sync_copy(Ph.at[pl.ds(eb, tch), pl.ds(qoff, QW)],
                        Pr4.at[0, pl.ds(0, tch)])
        ca = pltpu.async_copy(Ah.at[didxT.at[pl.ds(0, tch)]],
                              Ar4.at[0, pl.ds(0, tch)], sga.at[0])
        cb = pltpu.async_copy(Bh.at[sidxT.at[pl.ds(0, tch)]],
                              Br4.at[0, pl.ds(0, tch)], sga.at[0])
        ca.wait()
        cb.wait()
        compute_chunk(Ar4.at[0], Br4.at[0], Pr4.at[0], Mv4.at[0], tch)
        pltpu.sync_copy(Mv4.at[0, pl.ds(0, tch)],
                        Ssh.at[didxT.at[pl.ds(0, tch)]], add=True)

    plsc.subcore_barrier()
    for off, sz in BUFCH:
        pltpu.sync_copy(Ssh.at[pl.ds(base0 + off, sz)], buf.at[pl.ds(0, sz)])
        pltpu.sync_copy(buf.at[pl.ds(0, sz)], Sout3.at[sid, pl.ds(off, sz)])


def _edge_sc_call(Aqs, Bqs, ei2, P):
    def body(A0r, A1r, A2r, A3r, B0r, B1r, B2r, B3r, eir, Pr,
             S0r, S1r, S2r, S3r, didx4, sidx4, didxT, sidxT, Ar4, Br4, Pr4,
             Mv4, buf, Ssh, sld, sga, ssc):
        cid = lax.axis_index("c")
        sid = lax.axis_index("s")
        Aall = [A0r, A1r, A2r, A3r]
        Ball = [B0r, B1r, B2r, B3r]
        Sall = [S0r, S1r, S2r, S3r]
        for m in range(2):

            @pl.when(cid == 0)
            def _():
                q = 2 * m
                _edge_phase(sid, Aall[q], Ball[q], Pr, q * QW, Sall[q], eir,
                            didx4, sidx4, didxT, sidxT, Ar4, Br4, Pr4, Mv4,
                            buf, Ssh, sld, sga, ssc, eir)

            @pl.when(cid == 1)
            def _():
                q = 2 * m + 1
                _edge_phase(sid, Aall[q], Ball[q], Pr, q * QW, Sall[q], eir,
                            didx4, sidx4, didxT, sidxT, Ar4, Br4, Pr4, Mv4,
                            buf, Ssh, sld, sga, ssc, eir)

    f = pl.kernel(
        body,
        mesh=_mesh(),
        compiler_params=pltpu.CompilerParams(use_tc_tiling_on_sc=False, internal_scratch_in_bytes=262144),
        out_type=[jax.ShapeDtypeStruct((16, NP16, QW), jnp.float32)] * 4,
        scratch_types=[
            pltpu.VMEM((G, CH), jnp.int32),
            pltpu.VMEM((G, CH), jnp.int32),
            pltpu.VMEM((CH,), jnp.int32),
            pltpu.VMEM((CH,), jnp.int32),
            pltpu.VMEM((G, CH, QW), jnp.float32),
            pltpu.VMEM((G, CH, QW), jnp.float32),
            pltpu.VMEM((G, CH, QW), jnp.float32),
            pltpu.VMEM((G, CH, QW), jnp.float32),
            pltpu.VMEM((784, QW), jnp.float32),
            pltpu.VMEM_SHARED((NPAD, QW), jnp.float32),
            pltpu.SemaphoreType.DMA((G,)),
            pltpu.SemaphoreType.DMA((G,)),
            pltpu.SemaphoreType.DMA((G,)),
        ],
    )
    outs = f(Aqs[0], Aqs[1], Aqs[2], Aqs[3], Bqs[0], Bqs[1], Bqs[2], Bqs[3],
             ei2, P)
    return [o.reshape(NPAD, QW) for o in outs]


def _deg_sc_call(ei2):
    def body(eir, deg_r, didx4, didxT, onesv, buf, Dsh, sld):
        from jax.experimental.pallas import tpu_sc as plsc
        cid = lax.axis_index("c")
        sid = lax.axis_index("s")

        @pl.when(cid == 0)
        def _():
            one = jnp.full((16,), 1.0, jnp.float32)

            def ob(i, _):
                onesv[i, pl.ds(0, 16)] = one
                return 0
            lax.fori_loop(0, CH, ob, 0)
            _zero_rows(buf, 784)
            base0 = sid * NP16
            for off, sz in BUFCH:
                pltpu.sync_copy(buf.at[pl.ds(0, sz)], Dsh.at[pl.ds(base0 + off, sz)])
            plsc.subcore_barrier()
            tbase = sid * EPT

            def group_body(g, _):
                gbase = tbase + g * (G * CH)
                lds = []
                for p in range(G):
                    eb = gbase + p * CH
                    lds.append(pltpu.async_copy(eir.at[1, pl.ds(eb, CH)],
                                                didx4.at[p], sld.at[p]))
                for p in range(G):
                    lds[p].wait()
                    pltpu.sync_copy(onesv, Dsh.at[didx4.at[p]], add=True)
                return 0
            lax.fori_loop(0, NG, group_body, 0)

            toff = tbase + NG * G * CH
            for i, tch in enumerate(TCH):
                eb = toff + sum(TCH[:i])
                pltpu.sync_copy(eir.at[1, pl.ds(eb, tch)], didxT.at[pl.ds(0, tch)])
                pltpu.sync_copy(onesv.at[pl.ds(0, tch)],
                                Dsh.at[didxT.at[pl.ds(0, tch)]], add=True)
            plsc.subcore_barrier()

            for off, sz in BUFCH:
                pltpu.sync_copy(Dsh.at[pl.ds(base0 + off, sz)], buf.at[pl.ds(0, sz)])
                pltpu.sync_copy(buf.at[pl.ds(0, sz)], deg_r.at[sid, pl.ds(off, sz)])

    f = pl.kernel(
        body,
        mesh=_mesh(),
        compiler_params=pltpu.CompilerParams(use_tc_tiling_on_sc=False, internal_scratch_in_bytes=262144),
        out_type=[jax.ShapeDtypeStruct((16, NP16, 16), jnp.float32)],
        scratch_types=[
            pltpu.VMEM((G, CH), jnp.int32),
            pltpu.VMEM((CH,), jnp.int32),
            pltpu.VMEM((CH, 16), jnp.float32),
            pltpu.VMEM((784, 16), jnp.float32),
            pltpu.VMEM_SHARED((NPAD, 16), jnp.float32),
            pltpu.SemaphoreType.DMA((G,)),
        ],
    )
    return f(ei2)[0].reshape(NPAD, 16)


def kernel(x, edge_index, edge_attr, We1, be1, We2, be2, Wm1, bm1, Wm2, bm2,
           Wd1, bd1, Wd2, bd2, Wd3, bd3):
    ei2 = _prep_call(edge_index)

    Wabs = [jnp.concatenate([Wm1[l, :H, :], Wm1[l, H:2 * H, :]], axis=1)
            for l in range(L)]
    b1s = [jnp.concatenate([bm1[l], jnp.zeros((H,), jnp.float32)])[None, :]
           for l in range(L)]
    W1cs = [Wm1[l, 2 * H:, :] for l in range(L)]

    deg = _deg_sc_call(ei2)
    Ps = [_p_call(edge_attr, W1cs[l]) for l in range(L)]

    h, *AB = _enc_call(x, We1, be1[None, :], We2, be2[None, :], Wabs[0], b1s[0])
    for l in range(L):
        A = AB[:4]
        B = AB[4:]
        S = _edge_sc_call(A, B, ei2, Ps[l])
        if l < L - 1:
            h, *AB = _upd_proj_call(h, S, deg, Wm2[l], bm2[l][None, :],
                                    Wabs[l + 1], b1s[l + 1])
        else:
            out = _upd_dec_call(h, S, deg, Wm2[l], bm2[l][None, :],
                                Wd1, bd1[None, :], Wd2, bd2[None, :],
                                Wd3, bd3[None, :])
    return out


# speedup vs baseline: 1.0510x; 1.0510x over previous
import functools
import jax
import jax.numpy as jnp
from jax import lax
from jax.experimental import pallas as pl
from jax.experimental.pallas import tpu as pltpu

N = 50000
E = 800000
H = 64
QW = 16
L = 6
ROWS = 400
GRID = N // ROWS

CH = 128
G = 4
EPT = E // 16
NG = EPT // (CH * G)
TREM = EPT - NG * CH * G
TCH = [CH, CH, TREM - 2 * CH]
NP16 = 3128
NPAD = 16 * NP16
BUFCH = [(0, 784), (784, 784), (1568, 784), (2352, 776)]

_HI = lax.Precision.HIGHEST


def _row_spec(f):
    return pl.BlockSpec((ROWS, f), lambda i: (i, 0))


def _full_spec(shape):
    nd = len(shape)
    return pl.BlockSpec(shape, lambda i: (0,) * nd)


def _silu(v):
    return v * jax.nn.sigmoid(v)


def _q_outs():
    return ([_row_spec(QW)] * 8,
            [jax.ShapeDtypeStruct((N, QW), jnp.float32)] * 8)


def _split_proj(ab, out_refs):
    for q in range(8):
        out_refs[q][...] = ab[:, q * QW:(q + 1) * QW]


def _enc_body(x_ref, We1_ref, be1_ref, We2_ref, be2_ref, Wab_ref, b1_ref,
              h_ref, *q_refs):
    h1 = _silu(jnp.dot(x_ref[...], We1_ref[...],
                       preferred_element_type=jnp.float32, precision=_HI) + be1_ref[...])
    h = jnp.dot(h1, We2_ref[...], preferred_element_type=jnp.float32,
                precision=_HI) + be2_ref[...]
    h_ref[...] = h
    ab = jnp.dot(h, Wab_ref[...], preferred_element_type=jnp.float32,
                 precision=_HI) + b1_ref[...]
    _split_proj(ab, q_refs)


def _upd_proj_body(h_ref, S0, S1, S2, S3, deg_ref, W2_ref, b2_ref, Wab_ref,
                   b1_ref, h_ref_o, *q_refs):
    S = jnp.concatenate([S0[...], S1[...], S2[...], S3[...]], axis=-1)
    dcol = deg_ref[:, 0:1]
    h = h_ref[...] + jnp.dot(S, W2_ref[...], preferred_element_type=jnp.float32,
                             precision=_HI) + dcol * b2_ref[...]
    h_ref_o[...] = h
    ab = jnp.dot(h, Wab_ref[...], preferred_element_type=jnp.float32,
                 precision=_HI) + b1_ref[...]
    _split_proj(ab, q_refs)


def _upd_dec_body(h_ref, S0, S1, S2, S3, deg_ref, W2_ref, b2_ref, Wd1_ref,
                  bd1_ref, Wd2_ref, bd2_ref, Wd3_ref, bd3_ref, out_ref):
    S = jnp.concatenate([S0[...], S1[...], S2[...], S3[...]], axis=-1)
    dcol = deg_ref[:, 0:1]
    h = h_ref[...] + jnp.dot(S, W2_ref[...], preferred_element_type=jnp.float32,
                             precision=_HI) + dcol * b2_ref[...]
    o = _silu(jnp.dot(h, Wd1_ref[...], preferred_element_type=jnp.float32,
                      precision=_HI) + bd1_ref[...])
    o = _silu(jnp.dot(o, Wd2_ref[...], preferred_element_type=jnp.float32,
                      precision=_HI) + bd2_ref[...])
    out_ref[...] = jnp.dot(o, Wd3_ref[...], preferred_element_type=jnp.float32,
                           precision=_HI) + bd3_ref[...]


def _enc_call(x, We1, be1, We2, be2, Wab, b1):
    qspecs, qshapes = _q_outs()
    return pl.pallas_call(
        _enc_body,
        grid=(GRID,),
        in_specs=[_row_spec(14), _full_spec((14, H)), _full_spec((1, H)),
                  _full_spec((H, H)), _full_spec((1, H)),
                  _full_spec((H, 2 * H)), _full_spec((1, 2 * H))],
        out_specs=[_row_spec(H)] + qspecs,
        out_shape=[jax.ShapeDtypeStruct((N, H), jnp.float32)] + qshapes,
    )(x, We1, be1, We2, be2, Wab, b1)


def _upd_proj_call(h, S, deg, W2, b2, Wab, b1):
    qspecs, qshapes = _q_outs()
    return pl.pallas_call(
        _upd_proj_body,
        grid=(GRID,),
        in_specs=[_row_spec(H)] + [_row_spec(QW)] * 4 + [_row_spec(16),
                  _full_spec((H, H)), _full_spec((1, H)),
                  _full_spec((H, 2 * H)), _full_spec((1, 2 * H))],
        out_specs=[_row_spec(H)] + qspecs,
        out_shape=[jax.ShapeDtypeStruct((N, H), jnp.float32)] + qshapes,
    )(h, S[0], S[1], S[2], S[3], deg, W2, b2, Wab, b1)


def _upd_dec_call(h, S, deg, W2, b2, Wd1, bd1, Wd2, bd2, Wd3, bd3):
    return pl.pallas_call(
        _upd_dec_body,
        grid=(GRID,),
        in_specs=[_row_spec(H)] + [_row_spec(QW)] * 4 + [_row_spec(16),
                  _full_spec((H, H)), _full_spec((1, H)),
                  _full_spec((H, H)), _full_spec((1, H)),
                  _full_spec((H, H // 2)), _full_spec((1, H // 2)),
                  _full_spec((H // 2, 9)), _full_spec((1, 9))],
        out_specs=[_row_spec(9)],
        out_shape=[jax.ShapeDtypeStruct((N, 9), jnp.float32)],
    )(h, S[0], S[1], S[2], S[3], deg, W2, b2, Wd1, bd1, Wd2, bd2, Wd3, bd3)[0]


BE = 6400


def _prep_body(ei_ref, out_ref):
    out_ref[...] = ei_ref[...]


def _prep_call(edge_index):
    return pl.pallas_call(
        _prep_body,
        grid=(E // BE,),
        in_specs=[pl.BlockSpec((2, BE), lambda i: (0, i))],
        out_specs=[pl.BlockSpec((2, BE), lambda i: (0, i))],
        out_shape=[jax.ShapeDtypeStruct((2, E), jnp.int32)],
    )(edge_index)[0]


def _p_body(ea_ref, w_ref, p_ref):
    p_ref[...] = jnp.dot(ea_ref[...], w_ref[...],
                         preferred_element_type=jnp.float32, precision=_HI)


def _p_call(edge_attr, w):
    return pl.pallas_call(
        _p_body,
        grid=(E // BE,),
        in_specs=[pl.BlockSpec((BE, 3), lambda i: (i, 0)), _full_spec((3, H))],
        out_specs=[pl.BlockSpec((BE, H), lambda i: (i, 0))],
        out_shape=[jax.ShapeDtypeStruct((E, H), jnp.float32)],
    )(edge_attr, w)[0]


_sc_mesh = None


def _mesh():
    global _sc_mesh
    if _sc_mesh is None:
        from jax.experimental.pallas import tpu_sc as plsc
        _sc_mesh = plsc.VectorSubcoreMesh(core_axis_name="c", subcore_axis_name="s")
    return _sc_mesh


def _zero_rows(buf, nrows):
    z = jnp.zeros((16,), jnp.float32)

    def zb(i, _):
        buf[i, pl.ds(0, 16)] = z
        return 0
    lax.fori_loop(0, nrows, zb, 0)


def _edge_phase(sid, Ah, Bh, Ph, qoff, Sout3, ei2,
                didx4, sidx4, didxT, sidxT, Ar4, Br4, Pr4, Mv4, buf,
                Ssh, sld, sga, ssc, drain_src):
    from jax.experimental.pallas import tpu_sc as plsc
    _zero_rows(buf, 784)
    base0 = sid * NP16
    for off, sz in BUFCH:
        pltpu.sync_copy(buf.at[pl.ds(0, sz)], Ssh.at[pl.ds(base0 + off, sz)])
    plsc.subcore_barrier()

    tbase = sid * EPT

    def compute_chunk(Arr, Brr, Prr, Mvr, nch):
        def edge_body(e, _):
            pre = Arr[e, pl.ds(0, 16)] + Brr[e, pl.ds(0, 16)] + Prr[e, pl.ds(0, 16)]
            Mvr[e, pl.ds(0, 16)] = pre / (1.0 + jnp.exp(-pre))
            return 0
        lax.fori_loop(0, nch, edge_body, 0)

    def drain_sc(p):
        pltpu.make_async_copy(drain_src, Mv4.at[p], ssc.at[p]).wait()

    def group_body(g, _):
        gbase = tbase + g * (G * CH)
        lds = []
        for p in range(G):
            eb = gbase + p * CH

            @pl.when(g > 0)
            def _():
                drain_sc(p)
            c0 = pltpu.async_copy(ei2.at[1, pl.ds(eb, CH)], didx4.at[p], sld.at[p])
            c1 = pltpu.async_copy(ei2.at[0, pl.ds(eb, CH)], sidx4.at[p], sld.at[p])
            c2 = pltpu.async_copy(Ph.at[pl.ds(eb, CH), pl.ds(qoff, QW)],
                                  Pr4.at[p], sld.at[p])
            lds.append((c0, c1, c2))
        gas = []
        for p in range(G):
            for c in lds[p]:
                c.wait()
            ga = pltpu.async_copy(Ah.at[didx4.at[p]], Ar4.at[p], sga.at[p])
            gb = pltpu.async_copy(Bh.at[sidx4.at[p]], Br4.at[p], sga.at[p])
            gas.append((ga, gb))
        for p in range(G):
            for c in gas[p]:
                c.wait()
            compute_chunk(Ar4.at[p], Br4.at[p], Pr4.at[p], Mv4.at[p], CH)
            pltpu.async_copy(Mv4.at[p], Ssh.at[didx4.at[p]], ssc.at[p], add=True)
        return 0
    lax.fori_loop(0, NG, group_body, 0)
    for p in range(G):
        drain_sc(p)

    toff = tbase + NG * G * CH
    for i, tch in enumerate(TCH):
        eb = toff + sum(TCH[:i])
        pltpu.sync_copy(ei2.at[1, pl.ds(eb, tch)], didxT.at[pl.ds(0, tch)])
        pltpu.sync_copy(ei2.at[0, pl.ds(eb, tch)], sidxT.at[pl.ds(0, tch)])
        pltpu.sync_copy(Ph.at[pl.ds(eb, tch), pl.ds(qoff, QW)],
                        Pr4.at[0, pl.ds(0, tch)])
        ca = pltpu.async_copy(Ah.at[didxT.at[pl.ds(0, tch)]],
                              Ar4.at[0, pl.ds(0, tch)], sga.at[0])
        cb = pltpu.async_copy(Bh.at[sidxT.at[pl.ds(0, tch)]],
                              Br4.at[0, pl.ds(0, tch)], sga.at[0])
        ca.wait()
        cb.wait()
        compute_chunk(Ar4.at[0], Br4.at[0], Pr4.at[0], Mv4.at[0], tch)
        pltpu.sync_copy(Mv4.at[0, pl.ds(0, tch)],
                        Ssh.at[didxT.at[pl.ds(0, tch)]], add=True)

    plsc.subcore_barrier()
    for off, sz in BUFCH:
        pltpu.sync_copy(Ssh.at[pl.ds(base0 + off, sz)], buf.at[pl.ds(0, sz)])
        pltpu.sync_copy(buf.at[pl.ds(0, sz)], Sout3.at[sid, pl.ds(off, sz)])


def _edge_sc_call(m, Aq0, Aq1, Bq0, Bq1, ei2, P):
    qoff0 = 2 * m * QW
    qoff1 = (2 * m + 1) * QW

    def body(A0r, A1r, B0r, B1r, eir, Pr,
             S0r, S1r, didx4, sidx4, didxT, sidxT, Ar4, Br4, Pr4, Mv4,
             buf, Ssh, sld, sga, ssc):
        cid = lax.axis_index("c")
        sid = lax.axis_index("s")

        @pl.when(cid == 0)
        def _():
            _edge_phase(sid, A0r, B0r, Pr, qoff0, S0r, eir,
                        didx4, sidx4, didxT, sidxT, Ar4, Br4, Pr4, Mv4,
                        buf, Ssh, sld, sga, ssc, eir)

        @pl.when(cid == 1)
        def _():
            _edge_phase(sid, A1r, B1r, Pr, qoff1, S1r, eir,
                        didx4, sidx4, didxT, sidxT, Ar4, Br4, Pr4, Mv4,
                        buf, Ssh, sld, sga, ssc, eir)

    f = pl.kernel(
        body,
        mesh=_mesh(),
        compiler_params=pltpu.CompilerParams(use_tc_tiling_on_sc=False, internal_scratch_in_bytes=262144),
        out_type=[jax.ShapeDtypeStruct((16, NP16, QW), jnp.float32),
                  jax.ShapeDtypeStruct((16, NP16, QW), jnp.float32)],
        scratch_types=[
            pltpu.VMEM((G, CH), jnp.int32),
            pltpu.VMEM((G, CH), jnp.int32),
            pltpu.VMEM((CH,), jnp.int32),
            pltpu.VMEM((CH,), jnp.int32),
            pltpu.VMEM((G, CH, QW), jnp.float32),
            pltpu.VMEM((G, CH, QW), jnp.float32),
            pltpu.VMEM((G, CH, QW), jnp.float32),
            pltpu.VMEM((G, CH, QW), jnp.float32),
            pltpu.VMEM((784, QW), jnp.float32),
            pltpu.VMEM_SHARED((NPAD, QW), jnp.float32),
            pltpu.SemaphoreType.DMA((G,)),
            pltpu.SemaphoreType.DMA((G,)),
            pltpu.SemaphoreType.DMA((G,)),
        ],
    )
    S0, S1 = f(Aq0, Aq1, Bq0, Bq1, ei2, P)
    return S0.reshape(NPAD, QW), S1.reshape(NPAD, QW)


def _deg_sc_call(ei2):
    def body(eir, deg_r, didx4, didxT, onesv, buf, Dsh, sld):
        from jax.experimental.pallas import tpu_sc as plsc
        cid = lax.axis_index("c")
        sid = lax.axis_index("s")

        @pl.when(cid == 0)
        def _():
            one = jnp.full((16,), 1.0, jnp.float32)

            def ob(i, _):
                onesv[i, pl.ds(0, 16)] = one
                return 0
            lax.fori_loop(0, CH, ob, 0)
            _zero_rows(buf, 784)
            base0 = sid * NP16
            for off, sz in BUFCH:
                pltpu.sync_copy(buf.at[pl.ds(0, sz)], Dsh.at[pl.ds(base0 + off, sz)])
            plsc.subcore_barrier()
            tbase = sid * EPT

            def group_body(g, _):
                gbase = tbase + g * (G * CH)
                lds = []
                for p in range(G):
                    eb = gbase + p * CH
                    lds.append(pltpu.async_copy(eir.at[1, pl.ds(eb, CH)],
                                                didx4.at[p], sld.at[p]))
                for p in range(G):
                    lds[p].wait()
                    pltpu.sync_copy(onesv, Dsh.at[didx4.at[p]], add=True)
                return 0
            lax.fori_loop(0, NG, group_body, 0)

            toff = tbase + NG * G * CH
            for i, tch in enumerate(TCH):
                eb = toff + sum(TCH[:i])
                pltpu.sync_copy(eir.at[1, pl.ds(eb, tch)], didxT.at[pl.ds(0, tch)])
                pltpu.sync_copy(onesv.at[pl.ds(0, tch)],
                                Dsh.at[didxT.at[pl.ds(0, tch)]], add=True)
            plsc.subcore_barrier()

            for off, sz in BUFCH:
                pltpu.sync_copy(Dsh.at[pl.ds(base0 + off, sz)], buf.at[pl.ds(0, sz)])
                pltpu.sync_copy(buf.at[pl.ds(0, sz)], deg_r.at[sid, pl.ds(off, sz)])

    f = pl.kernel(
        body,
        mesh=_mesh(),
        compiler_params=pltpu.CompilerParams(use_tc_tiling_on_sc=False, internal_scratch_in_bytes=262144),
        out_type=[jax.ShapeDtypeStruct((16, NP16, 16), jnp.float32)],
        scratch_types=[
            pltpu.VMEM((G, CH), jnp.int32),
            pltpu.VMEM((CH,), jnp.int32),
            pltpu.VMEM((CH, 16), jnp.float32),
            pltpu.VMEM((784, 16), jnp.float32),
            pltpu.VMEM_SHARED((NPAD, 16), jnp.float32),
            pltpu.SemaphoreType.DMA((G,)),
        ],
    )
    return f(ei2)[0].reshape(NPAD, 16)


def kernel(x, edge_index, edge_attr, We1, be1, We2, be2, Wm1, bm1, Wm2, bm2,
           Wd1, bd1, Wd2, bd2, Wd3, bd3):
    ei2 = _prep_call(edge_index)

    Wabs = [jnp.concatenate([Wm1[l, :H, :], Wm1[l, H:2 * H, :]], axis=1)
            for l in range(L)]
    b1s = [jnp.concatenate([bm1[l], jnp.zeros((H,), jnp.float32)])[None, :]
           for l in range(L)]
    W1cs = [Wm1[l, 2 * H:, :] for l in range(L)]

    deg = _deg_sc_call(ei2)

    h, *AB = _enc_call(x, We1, be1[None, :], We2, be2[None, :], Wabs[0], b1s[0])
    for l in range(L):
        A = AB[:4]
        B = AB[4:]
        P = _p_call(edge_attr, W1cs[l])
        S = []
        for m in range(2):
            s0, s1 = _edge_sc_call(m, A[2 * m], A[2 * m + 1], B[2 * m],
                                   B[2 * m + 1], ei2, P)
            S += [s0, s1]
        if l < L - 1:
            h, *AB = _upd_proj_call(h, S, deg, Wm2[l], bm2[l][None, :],
                                    Wabs[l + 1], b1s[l + 1])
        else:
            out = _upd_dec_call(h, S, deg, Wm2[l], bm2[l][None, :],
                                Wd1, bd1[None, :], Wd2, bd2[None, :],
                                Wd3, bd3[None, :])
    return out
